# baseline (speedup 1.0000x reference)
import functools

import jax
import jax.numpy as jnp
from jax import lax
from jax.experimental import pallas as pl
from jax.experimental.pallas import tpu as pltpu
from jax.experimental.pallas import tpu_sc as plsc

F32 = jnp.float32
I32 = jnp.int32
K = 64


def _fps_body(pos_ref, out_ref, *, M):
    B, _, N = pos_ref.shape
    px = pos_ref[:, 0, :]
    py = pos_ref[:, 1, :]
    pz = pos_ref[:, 2, :]
    iota_n = lax.broadcasted_iota(I32, (B, N), 1)
    iota_m = lax.broadcasted_iota(I32, (B, M), 1)

    dx = px - px[:, :1]
    dy = py - py[:, :1]
    dz = pz - pz[:, :1]
    d0 = (dx * dx + dy * dy) + dz * dz

    selx0 = jnp.where(iota_m == 0, px[:, :1], 0.0)
    sely0 = jnp.where(iota_m == 0, py[:, :1], 0.0)
    selz0 = jnp.where(iota_m == 0, pz[:, :1], 0.0)

    def body(i, st):
        dists, sx, sy, sz = st
        m = jnp.max(dists, axis=1, keepdims=True)
        eq = dists == m
        nxt = jnp.min(jnp.where(eq, iota_n, N), axis=1, keepdims=True)
        selm = iota_n == nxt
        qx = jnp.sum(jnp.where(selm, px, 0.0), axis=1, keepdims=True)
        qy = jnp.sum(jnp.where(selm, py, 0.0), axis=1, keepdims=True)
        qz = jnp.sum(jnp.where(selm, pz, 0.0), axis=1, keepdims=True)
        ddx = px - qx
        ddy = py - qy
        ddz = pz - qz
        d = (ddx * ddx + ddy * ddy) + ddz * ddz
        dists = jnp.minimum(dists, d)
        put = iota_m == i
        sx = jnp.where(put, qx, sx)
        sy = jnp.where(put, qy, sy)
        sz = jnp.where(put, qz, sz)
        return (dists, sx, sy, sz)

    _, sx, sy, sz = lax.fori_loop(1, M, body, (d0, selx0, sely0, selz0))
    out_ref[:, 0, :] = sx
    out_ref[:, 1, :] = sy
    out_ref[:, 2, :] = sz


def _fps(pos_t, M):
    B, _, N = pos_t.shape
    return pl.pallas_call(
        functools.partial(_fps_body, M=M),
        out_shape=jax.ShapeDtypeStruct((B, 3, M), F32),
    )(pos_t)


def _iota16():
    return lax.broadcasted_iota(I32, (16,), 0)


def _splat(val, dtype=I32):
    return jnp.full((16,), val, dtype)


def _sc_search_gather_sa1(px, py, pz, qx, qy, qz, r2):
    B, N = px.shape
    BQ = qx.shape[0]
    Q = BQ // B
    NW = 32
    QPW = BQ // NW
    WPB = NW // B
    mesh = plsc.VectorSubcoreMesh(core_axis_name="c", subcore_axis_name="s")

    @functools.partial(
        pl.kernel,
        mesh=mesh,
        compiler_params=pltpu.CompilerParams(needs_layout_passes=False),
        out_type=[
            jax.ShapeDtypeStruct((BQ * K * 8,), F32),
            jax.ShapeDtypeStruct((BQ,), I32),
        ],
        scratch_types=[
            pltpu.VMEM((N,), F32),
            pltpu.VMEM((N,), F32),
            pltpu.VMEM((N,), F32),
            pltpu.VMEM((QPW,), F32),
            pltpu.VMEM((QPW,), F32),
            pltpu.VMEM((QPW,), F32),
            pltpu.VMEM((K * 8,), F32),
            pltpu.VMEM((QPW,), I32),
        ],
    )
    def body(px_h, py_h, pz_h, qx_h, qy_h, qz_h, feat_h, cnt_h,
             pxs, pys, pzs, qxs, qys, qzs, otile, cnts):
        wid = lax.axis_index("s") * 2 + lax.axis_index("c")
        b = wid // WPB
        qbase = wid * QPW
        pltpu.sync_copy(px_h.at[b], pxs)
        pltpu.sync_copy(py_h.at[b], pys)
        pltpu.sync_copy(pz_h.at[b], pzs)
        pltpu.sync_copy(qx_h.at[pl.ds(qbase, QPW)], qxs)
        pltpu.sync_copy(qy_h.at[pl.ds(qbase, QPW)], qys)
        pltpu.sync_copy(qz_h.at[pl.ds(qbase, QPW)], qzs)

        zf = jnp.zeros((16,), F32)
        for j in range(K * 8 // 16):
            otile[pl.ds(j * 16, 16)] = zf

        lanes = _iota16()

        def per_query(q, _):
            iq = _splat(q)
            qxv = plsc.load_gather(qxs, [iq])
            qyv = plsc.load_gather(qys, [iq])
            qzv = plsc.load_gather(qzs, [iq])

            def chunk(ci, cnt_v):
                base = ci * 16
                pxv = pxs[pl.ds(base, 16)]
                pyv = pys[pl.ds(base, 16)]
                pzv = pzs[pl.ds(base, 16)]
                dx = pxv - qxv
                dy = pyv - qyv
                dz = pzv - qzv
                d2 = (dx * dx + dy * dy) + dz * dz
                m = d2 <= r2
                slot = cnt_v + plsc.cumsum(m.astype(I32)) - 1
                ok = m & (slot < K)
                fidx = slot * 8
                plsc.store_scatter(otile, [fidx], pxv, mask=ok)
                plsc.store_scatter(otile, [fidx + 1], pyv, mask=ok)
                plsc.store_scatter(otile, [fidx + 2], pzv, mask=ok)
                plsc.store_scatter(otile, [fidx + 3], dx, mask=ok)
                plsc.store_scatter(otile, [fidx + 4], dy, mask=ok)
                plsc.store_scatter(otile, [fidx + 5], dz, mask=ok)
                return cnt_v + plsc.all_reduce_population_count(m)

            cnt_v = lax.fori_loop(0, N // 16, chunk, jnp.zeros((16,), I32))
            plsc.store_scatter(cnts, [iq], jnp.minimum(cnt_v, K),
                               mask=lanes == 0)
            pltpu.sync_copy(otile, feat_h.at[pl.ds((qbase + q) * K * 8,
                                                   K * 8)])
            return 0

        lax.fori_loop(0, QPW, per_query, 0)
        pltpu.sync_copy(cnts, cnt_h.at[pl.ds(qbase, QPW)])

    return body(px, py, pz, qx, qy, qz)


def _sc_search_gather_sa2(px, py, pz, qx, qy, qz, t1, r2):
    B, N = px.shape
    BQ = qx.shape[0]
    NW = 32
    QPW = BQ // NW
    WPB = NW // B
    D = t1.shape[1]
    mesh = plsc.VectorSubcoreMesh(core_axis_name="c", subcore_axis_name="s")

    @functools.partial(
        pl.kernel,
        mesh=mesh,
        compiler_params=pltpu.CompilerParams(needs_layout_passes=False),
        out_type=[
            jax.ShapeDtypeStruct((BQ, K, D), F32),
            jax.ShapeDtypeStruct((BQ,), I32),
        ],
        scratch_types=[
            pltpu.VMEM((N,), F32),
            pltpu.VMEM((N,), F32),
            pltpu.VMEM((N,), F32),
            pltpu.VMEM((QPW,), F32),
            pltpu.VMEM((QPW,), F32),
            pltpu.VMEM((QPW,), F32),
            pltpu.VMEM((2, K), I32),
            pltpu.VMEM((2, K, D), F32),
            pltpu.VMEM((QPW,), I32),
            pltpu.SemaphoreType.DMA,
            pltpu.SemaphoreType.DMA,
            pltpu.SemaphoreType.DMA,
            pltpu.SemaphoreType.DMA,
        ],
    )
    def body(px_h, py_h, pz_h, qx_h, qy_h, qz_h, t1_h, g_h, cnt_h,
             pxs, pys, pzs, qxs, qys, qzs, idxb, rows, cnts,
             sem0, sem1, osem0, osem1):
        wid = lax.axis_index("s") * 2 + lax.axis_index("c")
        b = wid // WPB
        qbase = wid * QPW
        pbase = b * N
        pltpu.sync_copy(px_h.at[b], pxs)
        pltpu.sync_copy(py_h.at[b], pys)
        pltpu.sync_copy(pz_h.at[b], pzs)
        pltpu.sync_copy(qx_h.at[pl.ds(qbase, QPW)], qxs)
        pltpu.sync_copy(qy_h.at[pl.ds(qbase, QPW)], qys)
        pltpu.sync_copy(qz_h.at[pl.ds(qbase, QPW)], qzs)

        zi = jnp.zeros((16,), I32)
        for buf in range(2):
            for j in range(K // 16):
                idxb.at[buf][pl.ds(j * 16, 16)] = zi + pbase

        lanes = _iota16()
        sems = (sem0, sem1)
        osems = (osem0, osem1)

        def scan_query(q, idx_ref):
            iq = _splat(q)
            qxv = plsc.load_gather(qxs, [iq])
            qyv = plsc.load_gather(qys, [iq])
            qzv = plsc.load_gather(qzs, [iq])

            def chunk(ci, cnt_v):
                base = ci * 16
                pxv = pxs[pl.ds(base, 16)]
                pyv = pys[pl.ds(base, 16)]
                pzv = pzs[pl.ds(base, 16)]
                dx = pxv - qxv
                dy = pyv - qyv
                dz = pzv - qzv
                d2 = (dx * dx + dy * dy) + dz * dz
                m = d2 <= r2
                slot = cnt_v + plsc.cumsum(m.astype(I32)) - 1
                ok = m & (slot < K)
                plsc.store_scatter(idx_ref, [slot],
                                   lanes + (base + pbase), mask=ok)
                return cnt_v + plsc.all_reduce_population_count(m)

            cnt_v = lax.fori_loop(0, N // 16, chunk, jnp.zeros((16,), I32))
            plsc.store_scatter(cnts, [iq], jnp.minimum(cnt_v, K),
                               mask=lanes == 0)

        def per_pair(p, _):
            for buf in range(2):
                q = p * 2 + buf
                idx_ref = idxb.at[buf]
                row_ref = rows.at[buf]

                @pl.when(p > 0)
                def _():
                    pltpu.make_async_copy(
                        t1_h.at[idx_ref], row_ref, sems[buf]).wait()
                    pltpu.async_copy(row_ref, g_h.at[qbase + q - 2],
                                     osems[buf])

                scan_query(q, idx_ref)

                @pl.when(p > 0)
                def _():
                    pltpu.make_async_copy(
                        row_ref, g_h.at[qbase], osems[buf]).wait()

                pltpu.async_copy(t1_h.at[idx_ref], row_ref, sems[buf])
            return 0

        lax.fori_loop(0, QPW // 2, per_pair, 0)
        for buf in range(2):
            pltpu.make_async_copy(
                t1_h.at[idxb.at[buf]], rows.at[buf], sems[buf]).wait()
            pltpu.sync_copy(rows.at[buf], g_h.at[qbase + QPW - 2 + buf])
        pltpu.sync_copy(cnts, cnt_h.at[pl.ds(qbase, QPW)])

    return body(px, py, pz, qx, qy, qz, t1)


def _mlp1_body(feat_ref, pen_ref, w0_ref, b0_ref, w1_ref, b1_ref, w2_ref,
               b2_ref, pos1_ref, wx2_ref, wr2_ref, x1_ref, t1_ref):
    QB = feat_ref.shape[0] // K
    X = feat_ref[...]
    h = jnp.dot(X, w0_ref[...], preferred_element_type=F32) + b0_ref[0:1, :]
    h = jnp.maximum(h, 0.0)
    h = jnp.dot(h, w1_ref[...], preferred_element_type=F32) + b1_ref[0:1, :]
    h = jnp.maximum(h, 0.0)
    h = jnp.dot(h, w2_ref[...], preferred_element_type=F32) + b2_ref[0:1, :]
    h = h + pen_ref[...]
    x1 = jnp.max(h.reshape(QB, K, 128), axis=1)
    x1_ref[...] = x1
    t1_ref[...] = (
        jnp.dot(x1, wx2_ref[...], preferred_element_type=F32)
        + jnp.dot(pos1_ref[...], wr2_ref[...], preferred_element_type=F32))


def _mlp2_body(g_ref, pen_ref, pos2_ref, wr2_ref, b0_ref, w1_ref, b1_ref,
               w2_ref, b2_ref, x2_ref):
    QB, _, D = g_ref.shape
    U = jnp.dot(pos2_ref[...], wr2_ref[...], preferred_element_type=F32)
    h = (g_ref[...]
         - lax.broadcast_in_dim(U, (QB, K, D), (0, 2))
         + lax.broadcast_in_dim(b0_ref[0:1, :], (QB, K, D), (1, 2)))
    h = jnp.maximum(h, 0.0).reshape(QB * K, D)
    h = jnp.dot(h, w1_ref[...], preferred_element_type=F32) + b1_ref[0:1, :]
    h = jnp.maximum(h, 0.0)
    h = jnp.dot(h, w2_ref[...], preferred_element_type=F32) + b2_ref[0:1, :]
    h = h + pen_ref[...]
    x2_ref[...] = jnp.max(h.reshape(QB, K, 256), axis=1)


def _stage3_body(x2_ref, pos2_ref, w3a_ref, w3b_ref, b0_ref, w1_ref, b1_ref,
                 w2_ref, b2_ref, wfc_ref, bfc_ref, out_ref):
    x2 = x2_ref[0]
    p = pos2_ref[0]
    h = (jnp.dot(x2, w3a_ref[...], preferred_element_type=F32)
         + jnp.dot(p, w3b_ref[...], preferred_element_type=F32)
         + b0_ref[0:1, :])
    h = jnp.maximum(h, 0.0)
    h = jnp.dot(h, w1_ref[...], preferred_element_type=F32) + b1_ref[0:1, :]
    h = jnp.maximum(h, 0.0)
    h = jnp.dot(h, w2_ref[...], preferred_element_type=F32) + b2_ref[0:1, :]
    g = jnp.max(h, axis=0, keepdims=True)
    out_ref[0] = jnp.dot(g, wfc_ref[...], preferred_element_type=F32) \
        + bfc_ref[0:1, :]


def _tile8(b):
    return jnp.tile(b.reshape(1, -1), (8, 1))


def _penalty(cnt):
    BQ = cnt.shape[0]
    kio = lax.broadcasted_iota(I32, (BQ, K), 1)
    pen = jnp.where(kio < cnt[:, None], 0.0, -jnp.inf).astype(F32)
    return pen.reshape(BQ * K, 1)


def _mlp1(feat, cnt, pos1p, W0p, b0, W1, b1, W2, b2, Wx2, Wr2p):
    BQ = cnt.shape[0]
    QB = 64
    grid = BQ // QB
    feat = feat.reshape(BQ * K, 8)
    full = lambda shp: pl.BlockSpec(shp, lambda i: (0, 0))
    return pl.pallas_call(
        _mlp1_body,
        grid=(grid,),
        in_specs=[
            pl.BlockSpec((QB * K, 8), lambda i: (i, 0)),
            pl.BlockSpec((QB * K, 1), lambda i: (i, 0)),
            full((8, 64)), full((8, 64)),
            full((64, 64)), full((8, 64)),
            full((64, 128)), full((8, 128)),
            pl.BlockSpec((QB, 8), lambda i: (i, 0)),
            full((128, 128)), full((8, 128)),
        ],
        out_specs=[
            pl.BlockSpec((QB, 128), lambda i: (i, 0)),
            pl.BlockSpec((QB, 128), lambda i: (i, 0)),
        ],
        out_shape=[
            jax.ShapeDtypeStruct((BQ, 128), F32),
            jax.ShapeDtypeStruct((BQ, 128), F32),
        ],
    )(feat, _penalty(cnt), W0p, _tile8(b0), W1, _tile8(b1), W2, _tile8(b2),
      pos1p, Wx2, Wr2p)


def _mlp2(G, cnt, pos2p, Wr2p, b0, W1, b1, W2, b2):
    BQ = G.shape[0]
    QB = 32
    grid = BQ // QB
    full = lambda shp: pl.BlockSpec(shp, lambda i: (0, 0))
    return pl.pallas_call(
        _mlp2_body,
        grid=(grid,),
        in_specs=[
            pl.BlockSpec((QB, K, 128), lambda i: (i, 0, 0)),
            pl.BlockSpec((QB * K, 1), lambda i: (i, 0)),
            pl.BlockSpec((QB, 8), lambda i: (i, 0)),
            full((8, 128)), full((8, 128)),
            full((128, 128)), full((8, 128)),
            full((128, 256)), full((8, 256)),
        ],
        out_specs=pl.BlockSpec((QB, 256), lambda i: (i, 0)),
        out_shape=jax.ShapeDtypeStruct((BQ, 256), F32),
    )(G, _penalty(cnt), pos2p, Wr2p, _tile8(b0), W1, _tile8(b1), W2,
      _tile8(b2))


def _stage3(x2, pos2p, W3a, W3bp, b0, W1, b1, W2, b2, Wfc, bfc):
    B = x2.shape[0]
    full = lambda shp: pl.BlockSpec(shp, lambda i: (0, 0))
    out = pl.pallas_call(
        _stage3_body,
        grid=(B,),
        in_specs=[
            pl.BlockSpec((1, 512, 256), lambda i: (i, 0, 0)),
            pl.BlockSpec((1, 512, 8), lambda i: (i, 0, 0)),
            full((256, 256)), full((8, 256)), full((8, 256)),
            full((256, 512)), full((8, 512)),
            full((512, 1024)), full((8, 1024)),
            full((1024, 256)), full((8, 256)),
        ],
        out_specs=pl.BlockSpec((1, 1, 256), lambda i: (i, 0, 0)),
        out_shape=jax.ShapeDtypeStruct((B, 1, 256), F32),
    )(x2, pos2p, W3a, W3bp, _tile8(b0), W1, _tile8(b1), W2, _tile8(b2),
      Wfc, _tile8(bfc))
    return out[:, 0, :]


def kernel(data, W1_0, b1_0, W1_1, b1_1, W1_2, b1_2, W2_0, b2_0, W2_1, b2_1,
           W2_2, b2_2, W3_0, b3_0, W3_1, b3_1, W3_2, b3_2, Wfc, bfc):
    B, N, _ = data.shape
    Q1, Q2 = N // 2, N // 8

    pos_t = jnp.transpose(data, (0, 2, 1))
    pos1_t = _fps(pos_t, Q1)
    pos2_t = _fps(pos1_t, Q2)

    q1 = [pos1_t[:, c, :].reshape(B * Q1) for c in range(3)]
    q2 = [pos2_t[:, c, :].reshape(B * Q2) for c in range(3)]
    p0 = [pos_t[:, c, :] for c in range(3)]
    p1 = [pos1_t[:, c, :] for c in range(3)]

    feat1, cnt1 = _sc_search_gather_sa1(*p0, *q1, F32(0.2 * 0.2))

    pos1p = jnp.pad(jnp.transpose(pos1_t, (0, 2, 1)).reshape(B * Q1, 3),
                    ((0, 0), (0, 5)))
    pos2p = jnp.pad(jnp.transpose(pos2_t, (0, 2, 1)).reshape(B * Q2, 3),
                    ((0, 0), (0, 5)))
    W0p = jnp.concatenate([W1_0, jnp.zeros((2, 64), F32)], axis=0)
    Wx2 = W2_0[:128]
    Wr2p = jnp.concatenate([W2_0[128:], jnp.zeros((5, 128), F32)], axis=0)

    x1, T1 = _mlp1(feat1, cnt1, pos1p, W0p, b1_0, W1_1, b1_1, W1_2, b1_2,
                   Wx2, Wr2p)

    G, cnt2 = _sc_search_gather_sa2(*p1, *q2, T1, F32(0.4 * 0.4))
    x2 = _mlp2(G, cnt2, pos2p, Wr2p, b2_0, W2_1, b2_1, W2_2, b2_2)

    W3a = W3_0[:256]
    W3bp = jnp.concatenate([W3_0[256:], jnp.zeros((5, 256), F32)], axis=0)
    out = _stage3(x2.reshape(B, Q2, 256), pos2p.reshape(B, Q2, 8),
                  W3a, W3bp, b3_0, W3_1, b3_1, W3_2, b3_2, Wfc, bfc)
    return out


# speedup vs baseline: 1.1567x; 1.1567x over previous
import functools

import jax
import jax.numpy as jnp
from jax import lax
from jax.experimental import pallas as pl
from jax.experimental.pallas import tpu as pltpu
from jax.experimental.pallas import tpu_sc as plsc

F32 = jnp.float32
I32 = jnp.int32
K = 64


def _fps_body(pos_ref, out_ref, *, M):
    B, _, N = pos_ref.shape
    px = pos_ref[:, 0, :]
    py = pos_ref[:, 1, :]
    pz = pos_ref[:, 2, :]
    iota_n = lax.broadcasted_iota(I32, (B, N), 1)
    iota_m = lax.broadcasted_iota(I32, (B, M), 1)

    dx = px - px[:, :1]
    dy = py - py[:, :1]
    dz = pz - pz[:, :1]
    d0 = (dx * dx + dy * dy) + dz * dz

    selx0 = jnp.where(iota_m == 0, px[:, :1], 0.0)
    sely0 = jnp.where(iota_m == 0, py[:, :1], 0.0)
    selz0 = jnp.where(iota_m == 0, pz[:, :1], 0.0)

    def body(i, st):
        dists, sx, sy, sz = st
        m = jnp.max(dists, axis=1, keepdims=True)
        eq = dists == m
        nxt = jnp.min(jnp.where(eq, iota_n, N), axis=1, keepdims=True)
        selm = iota_n == nxt
        qx = jnp.sum(jnp.where(selm, px, 0.0), axis=1, keepdims=True)
        qy = jnp.sum(jnp.where(selm, py, 0.0), axis=1, keepdims=True)
        qz = jnp.sum(jnp.where(selm, pz, 0.0), axis=1, keepdims=True)
        ddx = px - qx
        ddy = py - qy
        ddz = pz - qz
        d = (ddx * ddx + ddy * ddy) + ddz * ddz
        dists = jnp.minimum(dists, d)
        put = iota_m == i
        sx = jnp.where(put, qx, sx)
        sy = jnp.where(put, qy, sy)
        sz = jnp.where(put, qz, sz)
        return (dists, sx, sy, sz)

    _, sx, sy, sz = lax.fori_loop(1, M, body, (d0, selx0, sely0, selz0))
    out_ref[:, 0, :] = sx
    out_ref[:, 1, :] = sy
    out_ref[:, 2, :] = sz


def _fps(pos_t, M):
    B, _, N = pos_t.shape
    return pl.pallas_call(
        functools.partial(_fps_body, M=M),
        out_shape=jax.ShapeDtypeStruct((B, 3, M), F32),
    )(pos_t)


def _iota16():
    return lax.broadcasted_iota(I32, (16,), 0)


def _splat(val, dtype=I32):
    return jnp.full((16,), val, dtype)


def _sc_search_gather_sa1(px, py, pz, qx, qy, qz, r2):
    B, N = px.shape
    BQ = qx.shape[0]
    Q = BQ // B
    NW = 32
    QPW = BQ // NW
    WPB = NW // B
    mesh = plsc.VectorSubcoreMesh(core_axis_name="c", subcore_axis_name="s")

    @functools.partial(
        pl.kernel,
        mesh=mesh,
        compiler_params=pltpu.CompilerParams(needs_layout_passes=False),
        out_type=[
            jax.ShapeDtypeStruct((BQ * K * 8,), F32),
            jax.ShapeDtypeStruct((BQ,), I32),
        ],
        scratch_types=[
            pltpu.VMEM((N,), F32),
            pltpu.VMEM((N,), F32),
            pltpu.VMEM((N,), F32),
            pltpu.VMEM((QPW,), F32),
            pltpu.VMEM((QPW,), F32),
            pltpu.VMEM((QPW,), F32),
            pltpu.VMEM((K * 8,), F32),
            pltpu.VMEM((K * 8,), F32),
            pltpu.VMEM((QPW,), I32),
        ],
    )
    def body(px_h, py_h, pz_h, qx_h, qy_h, qz_h, feat_h, cnt_h,
             pxs, pys, pzs, qxs, qys, qzs, ot0, ot1, cnts):
        wid = lax.axis_index("s") * 2 + lax.axis_index("c")
        b = wid // WPB
        qbase = wid * QPW
        pltpu.sync_copy(px_h.at[b], pxs)
        pltpu.sync_copy(py_h.at[b], pys)
        pltpu.sync_copy(pz_h.at[b], pzs)
        pltpu.sync_copy(qx_h.at[pl.ds(qbase, QPW)], qxs)
        pltpu.sync_copy(qy_h.at[pl.ds(qbase, QPW)], qys)
        pltpu.sync_copy(qz_h.at[pl.ds(qbase, QPW)], qzs)

        zf = jnp.zeros((16,), F32)
        for j in range(K * 8 // 16):
            ot0[pl.ds(j * 16, 16)] = zf
            ot1[pl.ds(j * 16, 16)] = zf

        lanes = _iota16()

        def per_pair(p, _):
            q0 = p * 2
            iq0 = _splat(q0)
            iq1 = iq0 + 1
            qxv0 = plsc.load_gather(qxs, [iq0])
            qyv0 = plsc.load_gather(qys, [iq0])
            qzv0 = plsc.load_gather(qzs, [iq0])
            qxv1 = plsc.load_gather(qxs, [iq1])
            qyv1 = plsc.load_gather(qys, [iq1])
            qzv1 = plsc.load_gather(qzs, [iq1])

            def chunk(ci, carry):
                cnt0, cnt1 = carry
                base = ci * 16
                pxv = pxs[pl.ds(base, 16)]
                pyv = pys[pl.ds(base, 16)]
                pzv = pzs[pl.ds(base, 16)]
                dx0 = pxv - qxv0
                dy0 = pyv - qyv0
                dz0 = pzv - qzv0
                dx1 = pxv - qxv1
                dy1 = pyv - qyv1
                dz1 = pzv - qzv1
                d20 = (dx0 * dx0 + dy0 * dy0) + dz0 * dz0
                d21 = (dx1 * dx1 + dy1 * dy1) + dz1 * dz1
                m0 = d20 <= r2
                m1 = d21 <= r2
                s0 = cnt0 + plsc.cumsum(m0.astype(I32)) - 1
                s1 = cnt1 + plsc.cumsum(m1.astype(I32)) - 1
                ok0 = m0 & (s0 < K)
                ok1 = m1 & (s1 < K)
                f0 = s0 * 8
                f1 = s1 * 8
                plsc.store_scatter(ot0, [f0], pxv, mask=ok0)
                plsc.store_scatter(ot1, [f1], pxv, mask=ok1)
                plsc.store_scatter(ot0, [f0 + 1], pyv, mask=ok0)
                plsc.store_scatter(ot1, [f1 + 1], pyv, mask=ok1)
                plsc.store_scatter(ot0, [f0 + 2], pzv, mask=ok0)
                plsc.store_scatter(ot1, [f1 + 2], pzv, mask=ok1)
                plsc.store_scatter(ot0, [f0 + 3], dx0, mask=ok0)
                plsc.store_scatter(ot1, [f1 + 3], dx1, mask=ok1)
                plsc.store_scatter(ot0, [f0 + 4], dy0, mask=ok0)
                plsc.store_scatter(ot1, [f1 + 4], dy1, mask=ok1)
                plsc.store_scatter(ot0, [f0 + 5], dz0, mask=ok0)
                plsc.store_scatter(ot1, [f1 + 5], dz1, mask=ok1)
                return (cnt0 + plsc.all_reduce_population_count(m0),
                        cnt1 + plsc.all_reduce_population_count(m1))

            z16 = jnp.zeros((16,), I32)
            cnt0, cnt1 = lax.fori_loop(0, N // 16, chunk, (z16, z16))
            plsc.store_scatter(cnts, [iq0], jnp.minimum(cnt0, K),
                               mask=lanes == 0)
            plsc.store_scatter(cnts, [iq1], jnp.minimum(cnt1, K),
                               mask=lanes == 0)
            pltpu.sync_copy(ot0, feat_h.at[pl.ds((qbase + q0) * K * 8,
                                                 K * 8)])
            pltpu.sync_copy(ot1, feat_h.at[pl.ds((qbase + q0 + 1) * K * 8,
                                                 K * 8)])
            return 0

        lax.fori_loop(0, QPW // 2, per_pair, 0)
        pltpu.sync_copy(cnts, cnt_h.at[pl.ds(qbase, QPW)])

    return body(px, py, pz, qx, qy, qz)


def _sc_search_gather_sa2(px, py, pz, qx, qy, qz, t1, r2):
    B, N = px.shape
    BQ = qx.shape[0]
    NW = 32
    QPW = BQ // NW
    WPB = NW // B
    D = t1.shape[1]
    mesh = plsc.VectorSubcoreMesh(core_axis_name="c", subcore_axis_name="s")

    @functools.partial(
        pl.kernel,
        mesh=mesh,
        compiler_params=pltpu.CompilerParams(needs_layout_passes=False),
        out_type=[
            jax.ShapeDtypeStruct((BQ, K, D), F32),
            jax.ShapeDtypeStruct((BQ,), I32),
        ],
        scratch_types=[
            pltpu.VMEM((N,), F32),
            pltpu.VMEM((N,), F32),
            pltpu.VMEM((N,), F32),
            pltpu.VMEM((QPW,), F32),
            pltpu.VMEM((QPW,), F32),
            pltpu.VMEM((QPW,), F32),
            pltpu.VMEM((2, K), I32),
            pltpu.VMEM((2, K, D), F32),
            pltpu.VMEM((QPW,), I32),
            pltpu.SemaphoreType.DMA,
            pltpu.SemaphoreType.DMA,
            pltpu.SemaphoreType.DMA,
            pltpu.SemaphoreType.DMA,
        ],
    )
    def body(px_h, py_h, pz_h, qx_h, qy_h, qz_h, t1_h, g_h, cnt_h,
             pxs, pys, pzs, qxs, qys, qzs, idxb, rows, cnts,
             sem0, sem1, osem0, osem1):
        wid = lax.axis_index("s") * 2 + lax.axis_index("c")
        b = wid // WPB
        qbase = wid * QPW
        pbase = b * N
        pltpu.sync_copy(px_h.at[b], pxs)
        pltpu.sync_copy(py_h.at[b], pys)
        pltpu.sync_copy(pz_h.at[b], pzs)
        pltpu.sync_copy(qx_h.at[pl.ds(qbase, QPW)], qxs)
        pltpu.sync_copy(qy_h.at[pl.ds(qbase, QPW)], qys)
        pltpu.sync_copy(qz_h.at[pl.ds(qbase, QPW)], qzs)

        zi = jnp.zeros((16,), I32)
        for buf in range(2):
            for j in range(K // 16):
                idxb.at[buf][pl.ds(j * 16, 16)] = zi + pbase

        lanes = _iota16()
        sems = (sem0, sem1)
        osems = (osem0, osem1)

        def scan_query(q, idx_ref):
            iq = _splat(q)
            qxv = plsc.load_gather(qxs, [iq])
            qyv = plsc.load_gather(qys, [iq])
            qzv = plsc.load_gather(qzs, [iq])

            def chunk(ci, cnt_v):
                base = ci * 16
                pxv = pxs[pl.ds(base, 16)]
                pyv = pys[pl.ds(base, 16)]
                pzv = pzs[pl.ds(base, 16)]
                dx = pxv - qxv
                dy = pyv - qyv
                dz = pzv - qzv
                d2 = (dx * dx + dy * dy) + dz * dz
                m = d2 <= r2
                slot = cnt_v + plsc.cumsum(m.astype(I32)) - 1
                ok = m & (slot < K)
                plsc.store_scatter(idx_ref, [slot],
                                   lanes + (base + pbase), mask=ok)
                return cnt_v + plsc.all_reduce_population_count(m)

            cnt_v = lax.fori_loop(0, N // 16, chunk, jnp.zeros((16,), I32))
            plsc.store_scatter(cnts, [iq], jnp.minimum(cnt_v, K),
                               mask=lanes == 0)

        def per_pair(p, _):
            for buf in range(2):
                q = p * 2 + buf
                idx_ref = idxb.at[buf]
                row_ref = rows.at[buf]

                @pl.when(p > 0)
                def _():
                    pltpu.make_async_copy(
                        t1_h.at[idx_ref], row_ref, sems[buf]).wait()
                    pltpu.async_copy(row_ref, g_h.at[qbase + q - 2],
                                     osems[buf])

                scan_query(q, idx_ref)

                @pl.when(p > 0)
                def _():
                    pltpu.make_async_copy(
                        row_ref, g_h.at[qbase], osems[buf]).wait()

                pltpu.async_copy(t1_h.at[idx_ref], row_ref, sems[buf])
            return 0

        lax.fori_loop(0, QPW // 2, per_pair, 0)
        for buf in range(2):
            pltpu.make_async_copy(
                t1_h.at[idxb.at[buf]], rows.at[buf], sems[buf]).wait()
            pltpu.sync_copy(rows.at[buf], g_h.at[qbase + QPW - 2 + buf])
        pltpu.sync_copy(cnts, cnt_h.at[pl.ds(qbase, QPW)])

    return body(px, py, pz, qx, qy, qz, t1)


def _mlp1_body(feat_ref, pen_ref, w0_ref, b0_ref, w1_ref, b1_ref, w2_ref,
               b2_ref, pos1_ref, wx2_ref, wr2_ref, x1_ref, t1_ref):
    QB = feat_ref.shape[0] // K
    X = feat_ref[...]
    h = jnp.dot(X, w0_ref[...], preferred_element_type=F32) + b0_ref[0:1, :]
    h = jnp.maximum(h, 0.0)
    h = jnp.dot(h, w1_ref[...], preferred_element_type=F32) + b1_ref[0:1, :]
    h = jnp.maximum(h, 0.0)
    h = jnp.dot(h, w2_ref[...], preferred_element_type=F32) + b2_ref[0:1, :]
    h = h + pen_ref[...]
    x1 = jnp.max(h.reshape(QB, K, 128), axis=1)
    x1_ref[...] = x1
    t1_ref[...] = (
        jnp.dot(x1, wx2_ref[...], preferred_element_type=F32)
        + jnp.dot(pos1_ref[...], wr2_ref[...], preferred_element_type=F32))


def _mlp2_body(g_ref, pen_ref, pos2_ref, wr2_ref, b0_ref, w1_ref, b1_ref,
               w2_ref, b2_ref, x2_ref):
    QB, _, D = g_ref.shape
    U = jnp.dot(pos2_ref[...], wr2_ref[...], preferred_element_type=F32)
    h = (g_ref[...]
         - lax.broadcast_in_dim(U, (QB, K, D), (0, 2))
         + lax.broadcast_in_dim(b0_ref[0:1, :], (QB, K, D), (1, 2)))
    h = jnp.maximum(h, 0.0).reshape(QB * K, D)
    h = jnp.dot(h, w1_ref[...], preferred_element_type=F32) + b1_ref[0:1, :]
    h = jnp.maximum(h, 0.0)
    h = jnp.dot(h, w2_ref[...], preferred_element_type=F32) + b2_ref[0:1, :]
    h = h + pen_ref[...]
    x2_ref[...] = jnp.max(h.reshape(QB, K, 256), axis=1)


def _stage3_body(x2_ref, pos2_ref, w3a_ref, w3b_ref, b0_ref, w1_ref, b1_ref,
                 w2_ref, b2_ref, wfc_ref, bfc_ref, out_ref):
    x2 = x2_ref[0]
    p = pos2_ref[0]
    h = (jnp.dot(x2, w3a_ref[...], preferred_element_type=F32)
         + jnp.dot(p, w3b_ref[...], preferred_element_type=F32)
         + b0_ref[0:1, :])
    h = jnp.maximum(h, 0.0)
    h = jnp.dot(h, w1_ref[...], preferred_element_type=F32) + b1_ref[0:1, :]
    h = jnp.maximum(h, 0.0)
    h = jnp.dot(h, w2_ref[...], preferred_element_type=F32) + b2_ref[0:1, :]
    g = jnp.max(h, axis=0, keepdims=True)
    out_ref[0] = jnp.dot(g, wfc_ref[...], preferred_element_type=F32) \
        + bfc_ref[0:1, :]


def _tile8(b):
    return jnp.tile(b.reshape(1, -1), (8, 1))


def _penalty(cnt):
    BQ = cnt.shape[0]
    kio = lax.broadcasted_iota(I32, (BQ, K), 1)
    pen = jnp.where(kio < cnt[:, None], 0.0, -jnp.inf).astype(F32)
    return pen.reshape(BQ * K, 1)


def _mlp1(feat, cnt, pos1p, W0p, b0, W1, b1, W2, b2, Wx2, Wr2p):
    BQ = cnt.shape[0]
    QB = 64
    grid = BQ // QB
    feat = feat.reshape(BQ * K, 8)
    full = lambda shp: pl.BlockSpec(shp, lambda i: (0, 0))
    return pl.pallas_call(
        _mlp1_body,
        grid=(grid,),
        in_specs=[
            pl.BlockSpec((QB * K, 8), lambda i: (i, 0)),
            pl.BlockSpec((QB * K, 1), lambda i: (i, 0)),
            full((8, 64)), full((8, 64)),
            full((64, 64)), full((8, 64)),
            full((64, 128)), full((8, 128)),
            pl.BlockSpec((QB, 8), lambda i: (i, 0)),
            full((128, 128)), full((8, 128)),
        ],
        out_specs=[
            pl.BlockSpec((QB, 128), lambda i: (i, 0)),
            pl.BlockSpec((QB, 128), lambda i: (i, 0)),
        ],
        out_shape=[
            jax.ShapeDtypeStruct((BQ, 128), F32),
            jax.ShapeDtypeStruct((BQ, 128), F32),
        ],
    )(feat, _penalty(cnt), W0p, _tile8(b0), W1, _tile8(b1), W2, _tile8(b2),
      pos1p, Wx2, Wr2p)


def _mlp2(G, cnt, pos2p, Wr2p, b0, W1, b1, W2, b2):
    BQ = G.shape[0]
    QB = 32
    grid = BQ // QB
    full = lambda shp: pl.BlockSpec(shp, lambda i: (0, 0))
    return pl.pallas_call(
        _mlp2_body,
        grid=(grid,),
        in_specs=[
            pl.BlockSpec((QB, K, 128), lambda i: (i, 0, 0)),
            pl.BlockSpec((QB * K, 1), lambda i: (i, 0)),
            pl.BlockSpec((QB, 8), lambda i: (i, 0)),
            full((8, 128)), full((8, 128)),
            full((128, 128)), full((8, 128)),
            full((128, 256)), full((8, 256)),
        ],
        out_specs=pl.BlockSpec((QB, 256), lambda i: (i, 0)),
        out_shape=jax.ShapeDtypeStruct((BQ, 256), F32),
    )(G, _penalty(cnt), pos2p, Wr2p, _tile8(b0), W1, _tile8(b1), W2,
      _tile8(b2))


def _stage3(x2, pos2p, W3a, W3bp, b0, W1, b1, W2, b2, Wfc, bfc):
    B = x2.shape[0]
    full = lambda shp: pl.BlockSpec(shp, lambda i: (0, 0))
    out = pl.pallas_call(
        _stage3_body,
        grid=(B,),
        in_specs=[
            pl.BlockSpec((1, 512, 256), lambda i: (i, 0, 0)),
            pl.BlockSpec((1, 512, 8), lambda i: (i, 0, 0)),
            full((256, 256)), full((8, 256)), full((8, 256)),
            full((256, 512)), full((8, 512)),
            full((512, 1024)), full((8, 1024)),
            full((1024, 256)), full((8, 256)),
        ],
        out_specs=pl.BlockSpec((1, 1, 256), lambda i: (i, 0, 0)),
        out_shape=jax.ShapeDtypeStruct((B, 1, 256), F32),
    )(x2, pos2p, W3a, W3bp, _tile8(b0), W1, _tile8(b1), W2, _tile8(b2),
      Wfc, _tile8(bfc))
    return out[:, 0, :]


def kernel(data, W1_0, b1_0, W1_1, b1_1, W1_2, b1_2, W2_0, b2_0, W2_1, b2_1,
           W2_2, b2_2, W3_0, b3_0, W3_1, b3_1, W3_2, b3_2, Wfc, bfc):
    B, N, _ = data.shape
    Q1, Q2 = N // 2, N // 8

    pos_t = jnp.transpose(data, (0, 2, 1))
    pos1_t = _fps(pos_t, Q1)
    pos2_t = _fps(pos1_t, Q2)

    q1 = [pos1_t[:, c, :].reshape(B * Q1) for c in range(3)]
    q2 = [pos2_t[:, c, :].reshape(B * Q2) for c in range(3)]
    p0 = [pos_t[:, c, :] for c in range(3)]
    p1 = [pos1_t[:, c, :] for c in range(3)]

    feat1, cnt1 = _sc_search_gather_sa1(*p0, *q1, F32(0.2 * 0.2))

    pos1p = jnp.pad(jnp.transpose(pos1_t, (0, 2, 1)).reshape(B * Q1, 3),
                    ((0, 0), (0, 5)))
    pos2p = jnp.pad(jnp.transpose(pos2_t, (0, 2, 1)).reshape(B * Q2, 3),
                    ((0, 0), (0, 5)))
    W0p = jnp.concatenate([W1_0, jnp.zeros((2, 64), F32)], axis=0)
    Wx2 = W2_0[:128]
    Wr2p = jnp.concatenate([W2_0[128:], jnp.zeros((5, 128), F32)], axis=0)

    x1, T1 = _mlp1(feat1, cnt1, pos1p, W0p, b1_0, W1_1, b1_1, W1_2, b1_2,
                   Wx2, Wr2p)

    G, cnt2 = _sc_search_gather_sa2(*p1, *q2, T1, F32(0.4 * 0.4))
    x2 = _mlp2(G, cnt2, pos2p, Wr2p, b2_0, W2_1, b2_1, W2_2, b2_2)

    W3a = W3_0[:256]
    W3bp = jnp.concatenate([W3_0[256:], jnp.zeros((5, 256), F32)], axis=0)
    out = _stage3(x2.reshape(B, Q2, 256), pos2p.reshape(B, Q2, 8),
                  W3a, W3bp, b3_0, W3_1, b3_1, W3_2, b3_2, Wfc, bfc)
    return out
